# baseline (speedup 1.0000x reference)
import numpy as np
import jax
import jax.numpy as jnp
from jax.experimental import pallas as pl
from jax.experimental.pallas import tpu as pltpu


def _fused_kernel(x_ref, b1_ref, c2_ref, t2_ref, wf1_ref, bf1_ref, wf2_ref,
                  bf2_ref, wf3_ref, bf3_ref, out_ref):
    tb = x_ref.shape[0]
    rt = tb * 8
    r2 = rt - 2

    def mm(a, b):
        return jnp.dot(a, b, preferred_element_type=jnp.float32)

    def pool(z):
        return jnp.maximum(jnp.maximum(z[:, 0:128], z[:, 128:256]),
                           jnp.maximum(z[:, 256:384], z[:, 384:512]))

    lo, hi = [], []
    for c in range(3):
        xc = x_ref[:, c]
        lo.append(xc[:, :, 0:60].reshape(rt, 60))
        hi.append(xc[:, :, 60:120].reshape(rt, 60))
    z75 = jnp.zeros((rt, 75), jnp.bfloat16)
    ones = jnp.ones((rt, 1), jnp.bfloat16)
    x01 = jnp.concatenate(lo + [z75, ones], axis=1)
    x23 = jnp.concatenate(hi + [z75, ones], axis=1)

    z_e = mm(x01, b1_ref[0]) + mm(x23, b1_ref[1])
    z_o = mm(x23[0:rt - 1], b1_ref[0]) + mm(x01[1:rt], b1_ref[1])
    p_e = jnp.maximum(pool(z_e), 0.0).astype(jnp.bfloat16)
    p_o = jnp.maximum(pool(z_o), 0.0).astype(jnp.bfloat16)

    pk = jnp.concatenate([p_e[0:r2], p_o[0:r2]], axis=1)
    pk1 = jnp.concatenate([p_e[1:r2 + 1], p_o[1:r2 + 1]], axis=1)
    z2 = mm(pk, c2_ref[0]) + mm(pk1, c2_ref[1]) + t2_ref[...]
    p2 = jnp.maximum(pool(z2), 0.0)

    p2p = jnp.pad(p2, ((0, 2), (0, 0)))
    p3 = p2p.reshape(tb, 8, 128)
    acc = bf1_ref[...]
    for po in range(6):
        fpo = p3[:, po, :].astype(jnp.bfloat16)
        acc = acc + mm(fpo, wf1_ref[po])
    h1 = jnp.maximum(acc, 0.0).astype(jnp.bfloat16)

    h2 = jnp.maximum(mm(h1, wf2_ref[...]) + bf2_ref[...], 0.0).astype(jnp.bfloat16)
    out_ref[...] = mm(h2, wf3_ref[...]) + bf3_ref[...]


def _repad_kernel(x_ref, out_ref):
    xw = x_ref[...].astype(jnp.bfloat16)
    z = jnp.zeros((xw.shape[0], 60), jnp.bfloat16)
    out_ref[...] = jnp.concatenate(
        [xw[:, 0:900], z, xw[:, 900:1800], z, xw[:, 1800:2700], z], axis=1)


def _plane_perm():
    idx = np.arange(256)
    for c in range(3):
        for p in range(2):
            for w in range(30):
                idx[c * 60 + p * 30 + w] = p * 128 + w * 3 + c
    return idx


_PERM = _plane_perm()
_KEEP = np.zeros((256, 1), np.float32)
_KEEP[0:180] = 1.0


def _fold_params(b1, s1, t1, c2, s2):
    b1p = b1[:, _PERM, :].astype(jnp.float32) * s1[0][None, None, :]
    b1p = b1p * _KEEP[None]
    b1p = b1p.at[0, 255, :].set(t1[0])
    c2p = c2.astype(jnp.float32) * s2[0][None, None, :]
    return b1p.astype(jnp.bfloat16), c2p.astype(jnp.bfloat16)


@jax.jit
def _forward(b1, s1, t1, c2, s2, t2, wf1, bf1, wf2, bf2, wf3, bf3, x_nchw):
    n = x_nchw.shape[0]
    tb = min(128, max(8, ((n + 1) // 2 + 7) // 8 * 8))
    n_pad = ((n + tb - 1) // tb) * tb
    xb = x_nchw.reshape(n, 2700)
    if n_pad > n:
        xb = jnp.pad(xb, ((0, n_pad - n), (0, 0)))
    tb1 = 512 if n_pad % 512 == 0 else tb
    xp = pl.pallas_call(
        _repad_kernel,
        out_shape=jax.ShapeDtypeStruct((n_pad, 2880), jnp.bfloat16),
        grid=(n_pad // tb1,),
        in_specs=[pl.BlockSpec((tb1, 2700), lambda i: (i, 0))],
        out_specs=pl.BlockSpec((tb1, 2880), lambda i: (i, 0)),
        compiler_params=pltpu.CompilerParams(
            dimension_semantics=("parallel",)),
        cost_estimate=pl.CostEstimate(
            flops=n_pad * 2700, transcendentals=0,
            bytes_accessed=n_pad * 2700 * 4 + n_pad * 2880 * 2),
    )(xb)
    xb = xp.reshape(n_pad, 3, 8, 120)
    b1p, c2p = _fold_params(b1, s1, t1, c2, s2)
    wf1r = wf1.reshape(6, 128, 128)
    grid = (n_pad // tb,)

    c2d = lambda i: (0, 0)
    c3d = lambda i: (0, 0, 0)
    in_specs = [
        pl.BlockSpec((tb, 3, 8, 120), lambda i: (i, 0, 0, 0)),
        pl.BlockSpec((2, 256, 512), c3d),
        pl.BlockSpec((2, 256, 512), c3d),
        pl.BlockSpec((1, 512), c2d),
        pl.BlockSpec((6, 128, 128), c3d),
        pl.BlockSpec((1, 128), c2d),
        pl.BlockSpec((128, 128), c2d), pl.BlockSpec((1, 128), c2d),
        pl.BlockSpec((128, 128), c2d), pl.BlockSpec((1, 128), c2d),
    ]
    out_specs = pl.BlockSpec((tb, 128), lambda i: (i, 0))

    rt = tb * 8
    flops = grid[0] * 2 * (2 * rt * 256 * 512 + 2 * (rt - 1) * 256 * 512
                           + 2 * (rt - 2) * 256 * 512
                           + tb * (6 * 128 * 128 + 128 * 128 + 128 * 128))
    bytes_accessed = (n_pad * 2880 * 2 + n_pad * 128 * 4
                      + (4 * 256 * 512 + 768 * 128 + 2 * 128 * 128) * 2
                      + 6 * 128 * 4)

    out = pl.pallas_call(
        _fused_kernel,
        out_shape=jax.ShapeDtypeStruct((n_pad, 128), jnp.float32),
        grid=grid,
        in_specs=in_specs,
        out_specs=out_specs,
        compiler_params=pltpu.CompilerParams(
            dimension_semantics=("parallel",),
            vmem_limit_bytes=64 * 1024 * 1024),
        cost_estimate=pl.CostEstimate(flops=flops, transcendentals=0,
                                      bytes_accessed=bytes_accessed),
    )(xb, b1p, c2p, t2, wf1r, bf1, wf2, bf2, wf3, bf3)
    return out[:n, :10]


def kernel(b1, s1, t1, c2, s2, t2, wf1, bf1, wf2, bf2, wf3, bf3, x_nchw):
    return _forward(b1, s1, t1, c2, s2, t2, wf1, bf1, wf2, bf2, wf3, bf3,
                    x_nchw)


# speedup vs baseline: 1.3068x; 1.3068x over previous
import numpy as np
import jax
import jax.numpy as jnp
from jax.experimental import pallas as pl
from jax.experimental.pallas import tpu as pltpu


def _fused_kernel(x_ref, b1_ref, c2_ref, t2_ref, wf1_ref, bf1_ref, wf2_ref,
                  bf2_ref, wf3_ref, bf3_ref, out_ref):
    tb = x_ref.shape[0]
    rt = tb * 8
    r2 = rt - 2

    def mm(a, b):
        return jnp.dot(a, b, preferred_element_type=jnp.float32)

    def pool(z):
        return jnp.maximum(jnp.maximum(z[:, 0:128], z[:, 128:256]),
                           jnp.maximum(z[:, 256:384], z[:, 384:512]))

    lo, hi = [], []
    for c in range(3):
        xc = x_ref[:, c]
        lo.append(xc[:, :, 0:60].reshape(rt, 60))
        hi.append(xc[:, :, 60:120].reshape(rt, 60))
    z75 = jnp.zeros((rt, 75), jnp.bfloat16)
    ones = jnp.ones((rt, 1), jnp.bfloat16)
    x01 = jnp.concatenate(lo + [z75, ones], axis=1)
    x23 = jnp.concatenate(hi + [z75, ones], axis=1)

    z_e = mm(x01, b1_ref[0]) + mm(x23, b1_ref[1])
    z_o = mm(x23[0:rt - 1], b1_ref[0]) + mm(x01[1:rt], b1_ref[1])
    p_e = jnp.maximum(pool(z_e), 0.0).astype(jnp.bfloat16)
    p_o = jnp.maximum(pool(z_o), 0.0).astype(jnp.bfloat16)

    pk = jnp.concatenate([p_e[0:r2], p_o[0:r2]], axis=1)
    pk1 = jnp.concatenate([p_e[1:r2 + 1], p_o[1:r2 + 1]], axis=1)
    z2 = mm(pk, c2_ref[0]) + mm(pk1, c2_ref[1]) + t2_ref[...]
    p2 = jnp.maximum(pool(z2), 0.0)

    p2p = jnp.pad(p2, ((0, 2), (0, 0)))
    p3 = p2p.reshape(tb, 8, 128)
    acc = bf1_ref[...]
    for po in range(6):
        fpo = p3[:, po, :].astype(jnp.bfloat16)
        acc = acc + mm(fpo, wf1_ref[po])
    h1 = jnp.maximum(acc, 0.0).astype(jnp.bfloat16)

    h2 = jnp.maximum(mm(h1, wf2_ref[...]) + bf2_ref[...], 0.0).astype(jnp.bfloat16)
    out_ref[...] = (mm(h2, wf3_ref[...]) + bf3_ref[...])[:, 0:10]


def _repad_kernel(x_ref, out_ref):
    xw = x_ref[...].astype(jnp.bfloat16)
    z = jnp.zeros((xw.shape[0], 60), jnp.bfloat16)
    out_ref[...] = jnp.concatenate(
        [xw[:, 0:900], z, xw[:, 900:1800], z, xw[:, 1800:2700], z], axis=1)


def _plane_perm():
    idx = np.arange(256)
    for c in range(3):
        for p in range(2):
            for w in range(30):
                idx[c * 60 + p * 30 + w] = p * 128 + w * 3 + c
    return idx


_PERM = _plane_perm()
_KEEP = np.zeros((256, 1), np.float32)
_KEEP[0:180] = 1.0


def _fold_params(b1, s1, t1, c2, s2):
    b1p = b1[:, _PERM, :].astype(jnp.float32) * s1[0][None, None, :]
    b1p = b1p * _KEEP[None]
    b1p = b1p.at[0, 255, :].set(t1[0])
    c2p = c2.astype(jnp.float32) * s2[0][None, None, :]
    return b1p.astype(jnp.bfloat16), c2p.astype(jnp.bfloat16)


@jax.jit
def _forward(b1, s1, t1, c2, s2, t2, wf1, bf1, wf2, bf2, wf3, bf3, x_nchw):
    n = x_nchw.shape[0]
    tb = min(128, max(8, ((n + 1) // 2 + 7) // 8 * 8))
    n_pad = ((n + tb - 1) // tb) * tb
    xb = x_nchw.reshape(n, 3, 900).astype(jnp.bfloat16)
    xb = jnp.pad(xb, ((0, n_pad - n), (0, 0), (0, 60)))
    xb = xb.reshape(n_pad, 3, 8, 120)
    b1p, c2p = _fold_params(b1, s1, t1, c2, s2)
    wf1r = wf1.reshape(6, 128, 128)
    grid = (n_pad // tb,)

    c2d = lambda i: (0, 0)
    c3d = lambda i: (0, 0, 0)
    in_specs = [
        pl.BlockSpec((tb, 3, 8, 120), lambda i: (i, 0, 0, 0)),
        pl.BlockSpec((2, 256, 512), c3d),
        pl.BlockSpec((2, 256, 512), c3d),
        pl.BlockSpec((1, 512), c2d),
        pl.BlockSpec((6, 128, 128), c3d),
        pl.BlockSpec((1, 128), c2d),
        pl.BlockSpec((128, 128), c2d), pl.BlockSpec((1, 128), c2d),
        pl.BlockSpec((128, 128), c2d), pl.BlockSpec((1, 128), c2d),
    ]
    out_specs = pl.BlockSpec((tb, 10), lambda i: (i, 0))

    rt = tb * 8
    flops = grid[0] * 2 * (2 * rt * 256 * 512 + 2 * (rt - 1) * 256 * 512
                           + 2 * (rt - 2) * 256 * 512
                           + tb * (6 * 128 * 128 + 128 * 128 + 128 * 128))
    bytes_accessed = (n_pad * 2880 * 2 + n_pad * 128 * 4
                      + (4 * 256 * 512 + 768 * 128 + 2 * 128 * 128) * 2
                      + 6 * 128 * 4)

    out = pl.pallas_call(
        _fused_kernel,
        out_shape=jax.ShapeDtypeStruct((n_pad, 10), jnp.float32),
        grid=grid,
        in_specs=in_specs,
        out_specs=out_specs,
        compiler_params=pltpu.CompilerParams(
            dimension_semantics=("parallel",),
            vmem_limit_bytes=64 * 1024 * 1024),
        cost_estimate=pl.CostEstimate(flops=flops, transcendentals=0,
                                      bytes_accessed=bytes_accessed),
    )(xb, b1p, c2p, t2, wf1r, bf1, wf2, bf2, wf3, bf3)
    return out[:n]


def kernel(b1, s1, t1, c2, s2, t2, wf1, bf1, wf2, bf2, wf3, bf3, x_nchw):
    return _forward(b1, s1, t1, c2, s2, t2, wf1, bf1, wf2, bf2, wf3, bf3,
                    x_nchw)


# speedup vs baseline: 1.3602x; 1.0409x over previous
import numpy as np
import jax
import jax.numpy as jnp
from jax.experimental import pallas as pl
from jax.experimental.pallas import tpu as pltpu


def _fused_kernel(x_ref, b1_ref, c2_ref, t2_ref, wf1_ref, bf1_ref, wf2_ref,
                  bf2_ref, wf3_ref, bf3_ref, out_ref):
    tb = x_ref.shape[0]
    rt = tb * 8
    r2 = rt - 2

    def mm(a, b):
        return jnp.dot(a, b, preferred_element_type=jnp.float32)

    def pool(z):
        return jnp.maximum(jnp.maximum(z[:, 0:128], z[:, 128:256]),
                           jnp.maximum(z[:, 256:384], z[:, 384:512]))

    lo, hi = [], []
    for c in range(3):
        xc = x_ref[:, c]
        lo.append(xc[:, :, 0:60].reshape(rt, 60))
        hi.append(xc[:, :, 60:120].reshape(rt, 60))
    z75 = jnp.zeros((rt, 75), jnp.bfloat16)
    ones = jnp.ones((rt, 1), jnp.bfloat16)
    x01 = jnp.concatenate(lo + [z75, ones], axis=1)
    x23 = jnp.concatenate(hi + [z75, ones], axis=1)

    z_e = mm(x01, b1_ref[0]) + mm(x23, b1_ref[1])
    z_o = mm(x23[0:rt - 1], b1_ref[0]) + mm(x01[1:rt], b1_ref[1])
    p_e = jnp.maximum(pool(z_e), 0.0).astype(jnp.bfloat16)
    p_o = jnp.maximum(pool(z_o), 0.0).astype(jnp.bfloat16)

    pk = jnp.concatenate([p_e[0:r2], p_o[0:r2]], axis=1)
    pk1 = jnp.concatenate([p_e[1:r2 + 1], p_o[1:r2 + 1]], axis=1)
    z2 = mm(pk, c2_ref[0]) + mm(pk1, c2_ref[1]) + t2_ref[...]
    p2 = jnp.maximum(pool(z2), 0.0)

    p2p = jnp.pad(p2, ((0, 2), (0, 0)))
    p3 = p2p.reshape(tb, 8, 128)
    acc = bf1_ref[...]
    for po in range(6):
        fpo = p3[:, po, :].astype(jnp.bfloat16)
        acc = acc + mm(fpo, wf1_ref[po])
    h1 = jnp.maximum(acc, 0.0).astype(jnp.bfloat16)

    h2 = jnp.maximum(mm(h1, wf2_ref[...]) + bf2_ref[...], 0.0).astype(jnp.bfloat16)
    out_ref[...] = (mm(h2, wf3_ref[...]) + bf3_ref[...])[:, 0:10]


def _repad_kernel(x_ref, out_ref):
    xw = x_ref[...].astype(jnp.bfloat16)
    z = jnp.zeros((xw.shape[0], 60), jnp.bfloat16)
    out_ref[...] = jnp.concatenate(
        [xw[:, 0:900], z, xw[:, 900:1800], z, xw[:, 1800:2700], z], axis=1)


def _plane_perm():
    idx = np.arange(256)
    for c in range(3):
        for p in range(2):
            for w in range(30):
                idx[c * 60 + p * 30 + w] = p * 128 + w * 3 + c
    return idx


_PERM = _plane_perm()
_KEEP = np.zeros((256, 1), np.float32)
_KEEP[0:180] = 1.0


def _fold_params(b1, s1, t1, c2, s2):
    b1p = b1[:, _PERM, :].astype(jnp.float32) * s1[0][None, None, :]
    b1p = b1p * _KEEP[None]
    b1p = b1p.at[0, 255, :].set(t1[0])
    c2p = c2.astype(jnp.float32) * s2[0][None, None, :]
    return b1p.astype(jnp.bfloat16), c2p.astype(jnp.bfloat16)


@jax.jit
def _forward(b1, s1, t1, c2, s2, t2, wf1, bf1, wf2, bf2, wf3, bf3, x_nchw):
    n = x_nchw.shape[0]
    tb = min(256, max(8, ((n + 1) // 2 + 7) // 8 * 8))
    n_pad = ((n + tb - 1) // tb) * tb
    xb = x_nchw.reshape(n, 3, 900).astype(jnp.bfloat16)
    xb = jnp.pad(xb, ((0, n_pad - n), (0, 0), (0, 60)))
    xb = xb.reshape(n_pad, 3, 8, 120)
    b1p, c2p = _fold_params(b1, s1, t1, c2, s2)
    wf1r = wf1.reshape(6, 128, 128)
    grid = (n_pad // tb,)

    c2d = lambda i: (0, 0)
    c3d = lambda i: (0, 0, 0)
    in_specs = [
        pl.BlockSpec((tb, 3, 8, 120), lambda i: (i, 0, 0, 0)),
        pl.BlockSpec((2, 256, 512), c3d),
        pl.BlockSpec((2, 256, 512), c3d),
        pl.BlockSpec((1, 512), c2d),
        pl.BlockSpec((6, 128, 128), c3d),
        pl.BlockSpec((1, 128), c2d),
        pl.BlockSpec((128, 128), c2d), pl.BlockSpec((1, 128), c2d),
        pl.BlockSpec((128, 128), c2d), pl.BlockSpec((1, 128), c2d),
    ]
    out_specs = pl.BlockSpec((tb, 10), lambda i: (i, 0))

    rt = tb * 8
    flops = grid[0] * 2 * (2 * rt * 256 * 512 + 2 * (rt - 1) * 256 * 512
                           + 2 * (rt - 2) * 256 * 512
                           + tb * (6 * 128 * 128 + 128 * 128 + 128 * 128))
    bytes_accessed = (n_pad * 2880 * 2 + n_pad * 128 * 4
                      + (4 * 256 * 512 + 768 * 128 + 2 * 128 * 128) * 2
                      + 6 * 128 * 4)

    out = pl.pallas_call(
        _fused_kernel,
        out_shape=jax.ShapeDtypeStruct((n_pad, 10), jnp.float32),
        grid=grid,
        in_specs=in_specs,
        out_specs=out_specs,
        compiler_params=pltpu.CompilerParams(
            dimension_semantics=("parallel",),
            vmem_limit_bytes=64 * 1024 * 1024),
        cost_estimate=pl.CostEstimate(flops=flops, transcendentals=0,
                                      bytes_accessed=bytes_accessed),
    )(xb, b1p, c2p, t2, wf1r, bf1, wf2, bf2, wf3, bf3)
    return out[:n]


def kernel(b1, s1, t1, c2, s2, t2, wf1, bf1, wf2, bf2, wf3, bf3, x_nchw):
    return _forward(b1, s1, t1, c2, s2, t2, wf1, bf1, wf2, bf2, wf3, bf3,
                    x_nchw)


# speedup vs baseline: 1.3847x; 1.0180x over previous
import numpy as np
import jax
import jax.numpy as jnp
from jax.experimental import pallas as pl
from jax.experimental.pallas import tpu as pltpu


def _fused_kernel(x_ref, b1_ref, c2_ref, t2_ref, wf1_ref, bf1_ref, wf2_ref,
                  bf2_ref, wf3_ref, bf3_ref, out_ref):
    tb = x_ref.shape[0]
    rt = tb * 8
    r2 = rt - 2

    def mm(a, b):
        return jnp.dot(a, b, preferred_element_type=jnp.float32)

    def pool(z):
        return jnp.maximum(jnp.maximum(z[:, 0:128], z[:, 128:256]),
                           jnp.maximum(z[:, 256:384], z[:, 384:512]))

    lo, hi = [], []
    for c in range(3):
        xc = x_ref[:, c]
        lo.append(xc[:, :, 0:60].reshape(rt, 60))
        hi.append(xc[:, :, 60:120].reshape(rt, 60))
    z75 = jnp.zeros((rt, 75), jnp.bfloat16)
    ones = jnp.ones((rt, 1), jnp.bfloat16)
    x01 = jnp.concatenate(lo + [z75, ones], axis=1)
    x23 = jnp.concatenate(hi + [z75, ones], axis=1)

    z_e = mm(x01, b1_ref[0]) + mm(x23, b1_ref[1])
    z_o = mm(x23[0:rt - 1], b1_ref[0]) + mm(x01[1:rt], b1_ref[1])
    p_e = jnp.maximum(pool(z_e), 0.0).astype(jnp.bfloat16)
    p_o = jnp.maximum(pool(z_o), 0.0).astype(jnp.bfloat16)

    pk = jnp.concatenate([p_e[0:r2], p_o[0:r2]], axis=1)
    pk1 = jnp.concatenate([p_e[1:r2 + 1], p_o[1:r2 + 1]], axis=1)
    z2 = mm(pk, c2_ref[0]) + mm(pk1, c2_ref[1]) + t2_ref[...]
    p2 = jnp.maximum(pool(z2), 0.0)

    p2p = jnp.pad(p2, ((0, 2), (0, 0)))
    p3 = p2p.reshape(tb, 8, 128)
    acc = bf1_ref[...]
    for po in range(6):
        fpo = p3[:, po, :].astype(jnp.bfloat16)
        acc = acc + mm(fpo, wf1_ref[po])
    h1 = jnp.maximum(acc, 0.0).astype(jnp.bfloat16)

    h2 = jnp.maximum(mm(h1, wf2_ref[...]) + bf2_ref[...], 0.0).astype(jnp.bfloat16)
    out_ref[...] = (mm(h2, wf3_ref[...]) + bf3_ref[...])[:, 0:10]


def _repad_kernel(x_ref, out_ref):
    xw = x_ref[...].astype(jnp.bfloat16)
    z = jnp.zeros((xw.shape[0], 60), jnp.bfloat16)
    out_ref[...] = jnp.concatenate(
        [xw[:, 0:900], z, xw[:, 900:1800], z, xw[:, 1800:2700], z], axis=1)


def _plane_perm():
    idx = np.arange(256)
    for c in range(3):
        for p in range(2):
            for w in range(30):
                idx[c * 60 + p * 30 + w] = p * 128 + w * 3 + c
    return idx


_PERM = _plane_perm()
_KEEP = np.zeros((256, 1), np.float32)
_KEEP[0:180] = 1.0


def _fold_params(b1, s1, t1, c2, s2):
    b1p = b1[:, _PERM, :].astype(jnp.float32) * s1[0][None, None, :]
    b1p = b1p * _KEEP[None]
    b1p = b1p.at[0, 255, :].set(t1[0])
    c2p = c2.astype(jnp.float32) * s2[0][None, None, :]
    return b1p.astype(jnp.bfloat16), c2p.astype(jnp.bfloat16)


@jax.jit
def _forward(b1, s1, t1, c2, s2, t2, wf1, bf1, wf2, bf2, wf3, bf3, x_nchw):
    n = x_nchw.shape[0]
    tb = min(512, max(8, ((n + 1) // 2 + 7) // 8 * 8))
    n_pad = ((n + tb - 1) // tb) * tb
    xb = x_nchw.reshape(n, 3, 900).astype(jnp.bfloat16)
    xb = jnp.pad(xb, ((0, n_pad - n), (0, 0), (0, 60)))
    xb = xb.reshape(n_pad, 3, 8, 120)
    b1p, c2p = _fold_params(b1, s1, t1, c2, s2)
    wf1r = wf1.reshape(6, 128, 128)
    grid = (n_pad // tb,)

    c2d = lambda i: (0, 0)
    c3d = lambda i: (0, 0, 0)
    in_specs = [
        pl.BlockSpec((tb, 3, 8, 120), lambda i: (i, 0, 0, 0)),
        pl.BlockSpec((2, 256, 512), c3d),
        pl.BlockSpec((2, 256, 512), c3d),
        pl.BlockSpec((1, 512), c2d),
        pl.BlockSpec((6, 128, 128), c3d),
        pl.BlockSpec((1, 128), c2d),
        pl.BlockSpec((128, 128), c2d), pl.BlockSpec((1, 128), c2d),
        pl.BlockSpec((128, 128), c2d), pl.BlockSpec((1, 128), c2d),
    ]
    out_specs = pl.BlockSpec((tb, 10), lambda i: (i, 0))

    rt = tb * 8
    flops = grid[0] * 2 * (2 * rt * 256 * 512 + 2 * (rt - 1) * 256 * 512
                           + 2 * (rt - 2) * 256 * 512
                           + tb * (6 * 128 * 128 + 128 * 128 + 128 * 128))
    bytes_accessed = (n_pad * 2880 * 2 + n_pad * 128 * 4
                      + (4 * 256 * 512 + 768 * 128 + 2 * 128 * 128) * 2
                      + 6 * 128 * 4)

    out = pl.pallas_call(
        _fused_kernel,
        out_shape=jax.ShapeDtypeStruct((n_pad, 10), jnp.float32),
        grid=grid,
        in_specs=in_specs,
        out_specs=out_specs,
        compiler_params=pltpu.CompilerParams(
            dimension_semantics=("parallel",),
            vmem_limit_bytes=64 * 1024 * 1024),
        cost_estimate=pl.CostEstimate(flops=flops, transcendentals=0,
                                      bytes_accessed=bytes_accessed),
    )(xb, b1p, c2p, t2, wf1r, bf1, wf2, bf2, wf3, bf3)
    return out[:n]


def kernel(b1, s1, t1, c2, s2, t2, wf1, bf1, wf2, bf2, wf3, bf3, x_nchw):
    return _forward(b1, s1, t1, c2, s2, t2, wf1, bf1, wf2, bf2, wf3, bf3,
                    x_nchw)


# speedup vs baseline: 1.3895x; 1.0035x over previous
import numpy as np
import jax
import jax.numpy as jnp
from jax.experimental import pallas as pl
from jax.experimental.pallas import tpu as pltpu


def _fused_kernel(x_ref, b1_ref, c2_ref, t2_ref, wf1_ref, bf1_ref, wf2_ref,
                  bf2_ref, wf3_ref, bf3_ref, out_ref):
    tb = x_ref.shape[0]
    rt = tb * 8
    r2 = rt - 2

    def mm(a, b):
        return jnp.dot(a, b, preferred_element_type=jnp.float32)

    def pool(z):
        return jnp.maximum(jnp.maximum(z[:, 0:128], z[:, 128:256]),
                           jnp.maximum(z[:, 256:384], z[:, 384:512]))

    lo, hi = [], []
    for c in range(3):
        xc = x_ref[:, c]
        lo.append(xc[:, :, 0:60].reshape(rt, 60))
        hi.append(xc[:, :, 60:120].reshape(rt, 60))
    z75 = jnp.zeros((rt, 75), jnp.bfloat16)
    ones = jnp.ones((rt, 1), jnp.bfloat16)
    x01 = jnp.concatenate(lo + [z75, ones], axis=1)
    x23 = jnp.concatenate(hi + [z75, ones], axis=1)

    z_e = mm(x01, b1_ref[0]) + mm(x23, b1_ref[1])
    z_o = mm(x23[0:rt - 1], b1_ref[0]) + mm(x01[1:rt], b1_ref[1])
    p_e = jnp.maximum(pool(z_e), 0.0).astype(jnp.bfloat16)
    p_o = jnp.maximum(pool(z_o), 0.0).astype(jnp.bfloat16)

    pk = jnp.concatenate([p_e[0:r2], p_o[0:r2]], axis=1)
    pk1 = jnp.concatenate([p_e[1:r2 + 1], p_o[1:r2 + 1]], axis=1)
    z2 = mm(pk, c2_ref[0]) + mm(pk1, c2_ref[1]) + t2_ref[...]
    p2 = jnp.maximum(pool(z2), 0.0)

    p2p = jnp.pad(p2, ((0, 2), (0, 0)))
    p3 = p2p.reshape(tb, 8, 128)
    acc = bf1_ref[...]
    for po in range(6):
        fpo = p3[:, po, :].astype(jnp.bfloat16)
        acc = acc + mm(fpo, wf1_ref[po])
    h1 = jnp.maximum(acc, 0.0).astype(jnp.bfloat16)

    h2 = jnp.maximum(mm(h1, wf2_ref[...]) + bf2_ref[...], 0.0).astype(jnp.bfloat16)
    out_ref[...] = (mm(h2, wf3_ref[...]) + bf3_ref[...])[:, 0:10]


def _plane_perm():
    idx = np.arange(256)
    for c in range(3):
        for p in range(2):
            for w in range(30):
                idx[c * 60 + p * 30 + w] = p * 128 + w * 3 + c
    return idx


_PERM = _plane_perm()
_KEEP = np.zeros((256, 1), np.float32)
_KEEP[0:180] = 1.0


def _fold_params(b1, s1, t1, c2, s2):
    b1p = b1[:, _PERM, :].astype(jnp.float32) * s1[0][None, None, :]
    b1p = b1p * _KEEP[None]
    b1p = b1p.at[0, 255, :].set(t1[0])
    c2p = c2.astype(jnp.float32) * s2[0][None, None, :]
    return b1p.astype(jnp.bfloat16), c2p.astype(jnp.bfloat16)


@jax.jit
def _forward(b1, s1, t1, c2, s2, t2, wf1, bf1, wf2, bf2, wf3, bf3, x_nchw):
    n = x_nchw.shape[0]
    tb = min(512, max(8, ((n + 1) // 2 + 7) // 8 * 8))
    n_pad = ((n + tb - 1) // tb) * tb
    xb = x_nchw.reshape(n, 3, 900).astype(jnp.bfloat16)
    xb = jnp.pad(xb, ((0, n_pad - n), (0, 0), (0, 60)))
    xb = xb.reshape(n_pad, 3, 8, 120)
    b1p, c2p = _fold_params(b1, s1, t1, c2, s2)
    wf1r = wf1.reshape(6, 128, 128)
    grid = (n_pad // tb,)

    c2d = lambda i: (0, 0)
    c3d = lambda i: (0, 0, 0)
    in_specs = [
        pl.BlockSpec((tb, 3, 8, 120), lambda i: (i, 0, 0, 0)),
        pl.BlockSpec((2, 256, 512), c3d),
        pl.BlockSpec((2, 256, 512), c3d),
        pl.BlockSpec((1, 512), c2d),
        pl.BlockSpec((6, 128, 128), c3d),
        pl.BlockSpec((1, 128), c2d),
        pl.BlockSpec((128, 128), c2d), pl.BlockSpec((1, 128), c2d),
        pl.BlockSpec((128, 128), c2d), pl.BlockSpec((1, 128), c2d),
    ]
    out_specs = pl.BlockSpec((tb, 10), lambda i: (i, 0))

    rt = tb * 8
    flops = grid[0] * 2 * (2 * rt * 256 * 512 + 2 * (rt - 1) * 256 * 512
                           + 2 * (rt - 2) * 256 * 512
                           + tb * (6 * 128 * 128 + 128 * 128 + 128 * 128))
    bytes_accessed = (n_pad * 2880 * 2 + n_pad * 128 * 4
                      + (4 * 256 * 512 + 768 * 128 + 2 * 128 * 128) * 2
                      + 6 * 128 * 4)

    out = pl.pallas_call(
        _fused_kernel,
        out_shape=jax.ShapeDtypeStruct((n_pad, 10), jnp.float32),
        grid=grid,
        in_specs=in_specs,
        out_specs=out_specs,
        compiler_params=pltpu.CompilerParams(
            dimension_semantics=("parallel",),
            vmem_limit_bytes=100 * 1024 * 1024),
        cost_estimate=pl.CostEstimate(flops=flops, transcendentals=0,
                                      bytes_accessed=bytes_accessed),
    )(xb, b1p, c2p, t2, wf1r, bf1, wf2, bf2, wf3, bf3)
    return out[:n]


def kernel(b1, s1, t1, c2, s2, t2, wf1, bf1, wf2, bf2, wf3, bf3, x_nchw):
    return _forward(b1, s1, t1, c2, s2, t2, wf1, bf1, wf2, bf2, wf3, bf3,
                    x_nchw)
